# baseline (speedup 1.0000x reference)
import functools

import jax
import jax.numpy as jnp
from jax import lax
from jax.experimental import pallas as pl
from jax.experimental.pallas import tpu as pltpu
from jax.experimental.pallas import tpu_sc as plsc

B = 16384
F = 26
V = 100000
D = 32
BF = B * F

NC = 2
NS = 16
NW = NC * NS
RW = BF // NW
CH = 128
NCH = RW // CH


def _sc_gather(idx2d, tab2, fm2):
  mesh = plsc.VectorSubcoreMesh(core_axis_name="c", subcore_axis_name="s")

  @functools.partial(
      pl.kernel,
      mesh=mesh,
      compiler_params=pltpu.CompilerParams(use_tc_tiling_on_sc=False),
      out_type=[
          jax.ShapeDtypeStruct((BF, D), jnp.float32),
          jax.ShapeDtypeStruct((BF, 1), jnp.float32),
      ],
      scratch_types=[
          pltpu.VMEM((NCH, CH), jnp.int32),
          pltpu.VMEM((CH, D), jnp.float32),
          pltpu.VMEM((CH, 1), jnp.float32),
          pltpu.SemaphoreType.DMA,
          pltpu.SemaphoreType.DMA,
      ],
  )
  def k(idx_hbm, tab_hbm, fm_hbm, x_out, fm_out, idx_v, rows_v, fmrows_v,
        sem, sem2):
    wid = lax.axis_index("s") * NC + lax.axis_index("c")
    base = wid * RW
    pltpu.sync_copy(idx_hbm.at[pl.ds(wid * NCH, NCH)], idx_v)

    def body(j, _):
      a = pltpu.async_copy(tab_hbm.at[idx_v.at[j]], rows_v, sem)
      b = pltpu.async_copy(fm_hbm.at[idx_v.at[j]], fmrows_v, sem2)
      a.wait()
      b.wait()
      pltpu.sync_copy(rows_v, x_out.at[pl.ds(base + j * CH, CH)])
      pltpu.sync_copy(fmrows_v, fm_out.at[pl.ds(base + j * CH, CH)])
      return 0

    lax.fori_loop(0, NCH, body, 0)

  return k(idx2d, tab2, fm2)


def _dense_body(x_ref, fm_ref, w1, b1, w2, b2, w3, b3, w4, b4, out_ref):
  x = x_ref[...]
  fmv = fm_ref[...]
  r = lax.broadcasted_iota(jnp.int32, (F * D, D), 0)
  c = lax.broadcasted_iota(jnp.int32, (F * D, D), 1)
  smat = (r % D == c).astype(jnp.float32)
  s = jnp.dot(x, smat, preferred_element_type=jnp.float32)
  sq = jnp.sum(s * s, axis=1, keepdims=True)
  ss = jnp.sum(x * x, axis=1, keepdims=True)
  fm_second = 0.5 * (sq - ss)
  fm_first = jnp.sum(fmv, axis=1, keepdims=True)
  fm_output = fm_first + fm_second
  h = jax.nn.relu(jnp.dot(x, w1[...], preferred_element_type=jnp.float32)
                  + b1[...][None, :])
  h = jax.nn.relu(jnp.dot(h, w2[...], preferred_element_type=jnp.float32)
                  + b2[...][None, :])
  h = jax.nn.relu(jnp.dot(h, w3[...], preferred_element_type=jnp.float32)
                  + b3[...][None, :])
  deep = jax.nn.sigmoid(jnp.dot(h, w4[...], preferred_element_type=jnp.float32)
                        + b4[...][None, :])
  out_ref[...] = jax.nn.sigmoid(fm_output + deep)


def _tc_dense(x, fmv, W1, b1, W2, b2, W3, b3, W4, b4):
  BM = 2048
  grid = (B // BM,)
  return pl.pallas_call(
      _dense_body,
      grid=grid,
      in_specs=[
          pl.BlockSpec((BM, F * D), lambda i: (i, 0)),
          pl.BlockSpec((BM, F), lambda i: (i, 0)),
          pl.BlockSpec((F * D, 256), lambda i: (0, 0)),
          pl.BlockSpec((256,), lambda i: (0,)),
          pl.BlockSpec((256, 128), lambda i: (0, 0)),
          pl.BlockSpec((128,), lambda i: (0,)),
          pl.BlockSpec((128, 64), lambda i: (0, 0)),
          pl.BlockSpec((64,), lambda i: (0,)),
          pl.BlockSpec((64, 1), lambda i: (0, 0)),
          pl.BlockSpec((1,), lambda i: (0,)),
      ],
      out_specs=pl.BlockSpec((BM, 1), lambda i: (i, 0)),
      out_shape=jax.ShapeDtypeStruct((B, 1), jnp.float32),
  )(x, fmv, W1, b1, W2, b2, W3, b3, W4, b4)


def kernel(features, tables, fm_tables, W1, b1, W2, b2, W3, b3, W4, b4):
  offs = (jnp.arange(F, dtype=jnp.int32) * V)[None, :]
  flat = (features + offs).reshape(-1)
  tab2 = tables.reshape(F * V, D)
  fm2 = fm_tables.reshape(F * V, 1)
  x = jnp.take(tab2, flat, axis=0).reshape(B, F * D)
  fmv = jnp.take(fm2, flat, axis=0).reshape(B, F)
  return _tc_dense(x, fmv, W1, b1, W2, b2, W3, b3, W4, b4)


# speedup vs baseline: 8.4774x; 8.4774x over previous
import functools

import jax
import jax.numpy as jnp
from jax import lax
from jax.experimental import pallas as pl
from jax.experimental.pallas import tpu as pltpu
from jax.experimental.pallas import tpu_sc as plsc

B = 16384
F = 26
V = 100000
D = 32
BF = B * F

NC = 2
NS = 16
NW = NC * NS
RW = BF // NW
CH = 128
NCH = RW // CH


def _sc_gather(idx2d, tab2, fm2):
  mesh = plsc.VectorSubcoreMesh(core_axis_name="c", subcore_axis_name="s")

  @functools.partial(
      pl.kernel,
      mesh=mesh,
      compiler_params=pltpu.CompilerParams(use_tc_tiling_on_sc=False),
      out_type=[
          jax.ShapeDtypeStruct((BF, D), jnp.float32),
          jax.ShapeDtypeStruct((BF,), jnp.float32),
      ],
      scratch_types=[
          pltpu.VMEM((NCH, CH), jnp.int32),
          pltpu.VMEM((CH, D), jnp.float32),
          pltpu.VMEM((CH,), jnp.float32),
          pltpu.SemaphoreType.DMA,
          pltpu.SemaphoreType.DMA,
      ],
  )
  def k(idx_hbm, tab_hbm, fm_hbm, x_out, fm_out, idx_v, rows_v, fmrows_v,
        sem, sem2):
    wid = lax.axis_index("s") * NC + lax.axis_index("c")
    base = wid * RW
    pltpu.sync_copy(idx_hbm.at[pl.ds(wid * NCH, NCH)], idx_v)

    def body(j, _):
      a = pltpu.async_copy(tab_hbm.at[idx_v.at[j]], rows_v, sem)
      b = pltpu.async_copy(fm_hbm.at[idx_v.at[j]], fmrows_v, sem2)
      a.wait()
      b.wait()
      pltpu.sync_copy(rows_v, x_out.at[pl.ds(base + j * CH, CH)])
      pltpu.sync_copy(fmrows_v, fm_out.at[pl.ds(base + j * CH, CH)])
      return 0

    lax.fori_loop(0, NCH, body, 0)

  return k(idx2d, tab2, fm2)


def _dense_body(x_ref, fm_ref, w1, b1, w2, b2, w3, b3, w4, b4, out_ref):
  x = x_ref[...]
  fmv = fm_ref[...]
  r = lax.broadcasted_iota(jnp.int32, (F * D, D), 0)
  c = lax.broadcasted_iota(jnp.int32, (F * D, D), 1)
  smat = (r % D == c).astype(jnp.float32)
  s = jnp.dot(x, smat, preferred_element_type=jnp.float32)
  sq = jnp.sum(s * s, axis=1, keepdims=True)
  ss = jnp.sum(x * x, axis=1, keepdims=True)
  fm_second = 0.5 * (sq - ss)
  fm_first = jnp.sum(fmv, axis=1, keepdims=True)
  fm_output = fm_first + fm_second
  h = jax.nn.relu(jnp.dot(x, w1[...], preferred_element_type=jnp.float32)
                  + b1[...][None, :])
  h = jax.nn.relu(jnp.dot(h, w2[...], preferred_element_type=jnp.float32)
                  + b2[...][None, :])
  h = jax.nn.relu(jnp.dot(h, w3[...], preferred_element_type=jnp.float32)
                  + b3[...][None, :])
  deep = jax.nn.sigmoid(jnp.dot(h, w4[...], preferred_element_type=jnp.float32)
                        + b4[...][None, :])
  out_ref[...] = jax.nn.sigmoid(fm_output + deep)


def _tc_dense(x, fmv, W1, b1, W2, b2, W3, b3, W4, b4):
  BM = 2048
  grid = (B // BM,)
  return pl.pallas_call(
      _dense_body,
      grid=grid,
      in_specs=[
          pl.BlockSpec((BM, F * D), lambda i: (i, 0)),
          pl.BlockSpec((BM, F), lambda i: (i, 0)),
          pl.BlockSpec((F * D, 256), lambda i: (0, 0)),
          pl.BlockSpec((256,), lambda i: (0,)),
          pl.BlockSpec((256, 128), lambda i: (0, 0)),
          pl.BlockSpec((128,), lambda i: (0,)),
          pl.BlockSpec((128, 64), lambda i: (0, 0)),
          pl.BlockSpec((64,), lambda i: (0,)),
          pl.BlockSpec((64, 1), lambda i: (0, 0)),
          pl.BlockSpec((1,), lambda i: (0,)),
      ],
      out_specs=pl.BlockSpec((BM, 1), lambda i: (i, 0)),
      out_shape=jax.ShapeDtypeStruct((B, 1), jnp.float32),
  )(x, fmv, W1, b1, W2, b2, W3, b3, W4, b4)


def kernel(features, tables, fm_tables, W1, b1, W2, b2, W3, b3, W4, b4):
  offs = (jnp.arange(F, dtype=jnp.int32) * V)[None, :]
  idx2d = (features + offs).reshape(BF // CH, CH)
  tab2 = tables.reshape(F * V, D)
  fm2 = fm_tables.reshape(F * V)
  x_flat, fm_flat = _sc_gather(idx2d, tab2, fm2)
  x = x_flat.reshape(B, F * D)
  fmv = fm_flat.reshape(B, F)
  return _tc_dense(x, fmv, W1, b1, W2, b2, W3, b3, W4, b4)
